# Initial kernel scaffold; baseline (speedup 1.0000x reference)
import jax
import jax.numpy as jnp
from jax.experimental import pallas as pl


def kernel(edge_index, emb, W1, b1, W2, b2):
    raise NotImplementedError("write your pallas kernel here")


# speedup vs baseline: 17.5310x; 17.5310x over previous
import functools

import jax
import jax.numpy as jnp
from jax import lax
from jax.experimental import pallas as pl
from jax.experimental.pallas import tpu as pltpu
from jax.experimental.pallas import tpu_sc as plsc

F32 = jnp.float32
_NC = 2
_NS = 16
_NW = _NC * _NS


def _edge_chunks(E):
    per_w = E // _NW
    ch = max(c for c in range(8, 129, 8) if per_w % c == 0)
    return per_w, ch, per_w // ch


def _make_deg(N, E):
    W = 128
    per_w, ch, nch = _edge_chunks(E)
    rps = N // _NS
    mesh = plsc.VectorSubcoreMesh(core_axis_name="c", subcore_axis_name="s")

    @functools.partial(
        pl.kernel,
        out_type=jax.ShapeDtypeStruct((_NW, rps, W), F32),
        mesh=mesh,
        scratch_types=[
            pltpu.VMEM((nch, ch), jnp.int32),
            pltpu.VMEM((ch, W), F32),
            pltpu.VMEM_SHARED((N, W), F32),
        ],
    )
    def deg_kernel(dst_hbm, ones_hbm, zeros_hbm, out_hbm, idx_v, ones_v, acc_sh):
        cid = lax.axis_index("c")
        sid = lax.axis_index("s")
        wid = cid * _NS + sid
        pltpu.sync_copy(dst_hbm.at[wid], idx_v)
        pltpu.sync_copy(ones_hbm, ones_v)
        pltpu.sync_copy(zeros_hbm, acc_sh.at[pl.ds(sid * rps, rps)])
        plsc.subcore_barrier()

        def body(j, carry):
            pltpu.sync_copy(ones_v, acc_sh.at[idx_v.at[j]], add=True)
            return carry

        lax.fori_loop(0, nch, body, 0)
        plsc.subcore_barrier()
        pltpu.sync_copy(acc_sh.at[pl.ds(sid * rps, rps)], out_hbm.at[wid])

    return deg_kernel


def _make_agg(N, D, E):
    per_w, ch, nch = _edge_chunks(E)
    rps = N // _NS
    mesh = plsc.VectorSubcoreMesh(core_axis_name="c", subcore_axis_name="s")

    @functools.partial(
        pl.kernel,
        out_type=jax.ShapeDtypeStruct((_NW, rps, D), F32),
        mesh=mesh,
        scratch_types=[
            pltpu.VMEM((nch, ch), jnp.int32),
            pltpu.VMEM((nch, ch), jnp.int32),
            pltpu.VMEM((ch, D), F32),
            pltpu.VMEM_SHARED((N, D), F32),
            pltpu.SemaphoreType.DMA,
        ],
    )
    def agg_kernel(x_hbm, src_hbm, dst_hbm, zeros_hbm, out_hbm,
                   sidx_v, didx_v, rows_v, acc_sh, sem):
        cid = lax.axis_index("c")
        sid = lax.axis_index("s")
        wid = cid * _NS + sid
        pltpu.sync_copy(src_hbm.at[wid], sidx_v)
        pltpu.sync_copy(dst_hbm.at[wid], didx_v)
        pltpu.sync_copy(zeros_hbm, acc_sh.at[pl.ds(sid * rps, rps)])
        plsc.subcore_barrier()

        def body(j, carry):
            pltpu.async_copy(x_hbm.at[sidx_v.at[j]], rows_v, sem).wait()
            pltpu.sync_copy(rows_v, acc_sh.at[didx_v.at[j]], add=True)
            return carry

        lax.fori_loop(0, nch, body, 0)
        plsc.subcore_barrier()
        pltpu.sync_copy(acc_sh.at[pl.ds(sid * rps, rps)], out_hbm.at[wid])

    return agg_kernel


def _row_block(N):
    return max(r for r in range(8, 513, 8) if N % r == 0)


def _prescale(p0, p1, emb):
    N, D = emb.shape
    R = _row_block(N)

    def body(p0_ref, p1_ref, emb_ref, xs_ref, dis_ref):
        deg = p0_ref[:, 0:1] + p1_ref[:, 0:1] + 1.0
        dis = lax.rsqrt(deg)
        dis_ref[...] = dis
        xs_ref[...] = emb_ref[...] * dis

    return pl.pallas_call(
        body,
        grid=(N // R,),
        in_specs=[
            pl.BlockSpec((R, 128), lambda i: (i, 0)),
            pl.BlockSpec((R, 128), lambda i: (i, 0)),
            pl.BlockSpec((R, D), lambda i: (i, 0)),
        ],
        out_specs=[
            pl.BlockSpec((R, D), lambda i: (i, 0)),
            pl.BlockSpec((R, 1), lambda i: (i, 0)),
        ],
        out_shape=[
            jax.ShapeDtypeStruct((N, D), F32),
            jax.ShapeDtypeStruct((N, 1), F32),
        ],
    )(p0, p1, emb)


def _dense_mid(a0, a1, xs, dis, W1, b1, W2):
    N, D = xs.shape
    H = W1.shape[1]
    R = _row_block(N)

    def body(a0_ref, a1_ref, xs_ref, dis_ref, W1_ref, b1_ref, W2_ref, out_ref):
        pre = (a0_ref[...] + a1_ref[...] + xs_ref[...]) * dis_ref[...]
        h1 = jnp.dot(pre, W1_ref[...], preferred_element_type=F32) + b1_ref[...]
        ss = jnp.sum(h1 * h1, axis=1, keepdims=True)
        h = h1 / jnp.maximum(jnp.sqrt(ss), 1e-12)
        x2 = jnp.dot(h, W2_ref[...], preferred_element_type=F32)
        out_ref[...] = x2 * dis_ref[...]

    return pl.pallas_call(
        body,
        grid=(N // R,),
        in_specs=[
            pl.BlockSpec((R, D), lambda i: (i, 0)),
            pl.BlockSpec((R, D), lambda i: (i, 0)),
            pl.BlockSpec((R, D), lambda i: (i, 0)),
            pl.BlockSpec((R, 1), lambda i: (i, 0)),
            pl.BlockSpec((D, H), lambda i: (0, 0)),
            pl.BlockSpec((1, H), lambda i: (0, 0)),
            pl.BlockSpec((H, D), lambda i: (0, 0)),
        ],
        out_specs=pl.BlockSpec((R, D), lambda i: (i, 0)),
        out_shape=jax.ShapeDtypeStruct((N, D), F32),
    )(a0, a1, xs, dis, W1, b1, W2)


def _final(q0, q1, x2s, dis, b2):
    N, D = x2s.shape
    R = _row_block(N)

    def body(q0_ref, q1_ref, x2s_ref, dis_ref, b2_ref, out_ref):
        out_ref[...] = (q0_ref[...] + q1_ref[...] + x2s_ref[...]) * dis_ref[...] + b2_ref[...]

    return pl.pallas_call(
        body,
        grid=(N // R,),
        in_specs=[
            pl.BlockSpec((R, D), lambda i: (i, 0)),
            pl.BlockSpec((R, D), lambda i: (i, 0)),
            pl.BlockSpec((R, D), lambda i: (i, 0)),
            pl.BlockSpec((R, 1), lambda i: (i, 0)),
            pl.BlockSpec((1, D), lambda i: (0, 0)),
        ],
        out_specs=pl.BlockSpec((R, D), lambda i: (i, 0)),
        out_shape=jax.ShapeDtypeStruct((N, D), F32),
    )(q0, q1, x2s, dis, b2)


def kernel(edge_index, emb, W1, b1, W2, b2):
    N, D = emb.shape
    E = edge_index.shape[1]
    per_w, ch, nch = _edge_chunks(E)
    rps = N // _NS

    src3 = edge_index[0].astype(jnp.int32).reshape(_NW, nch, ch)
    dst3 = edge_index[1].astype(jnp.int32).reshape(_NW, nch, ch)
    ones128 = jnp.ones((ch, 128), F32)
    zeros128 = jnp.zeros((rps, 128), F32)
    zerosD = jnp.zeros((rps, D), F32)

    degp = _make_deg(N, E)(dst3, ones128, zeros128).reshape(_NC, N, 128)
    xs, dis = _prescale(degp[0], degp[1], emb)

    agg_fn = _make_agg(N, D, E)
    a = agg_fn(xs, src3, dst3, zerosD).reshape(_NC, N, D)
    x2s = _dense_mid(a[0], a[1], xs, dis, W1, b1.reshape(1, -1), W2)
    q = agg_fn(x2s, src3, dst3, zerosD).reshape(_NC, N, D)
    return _final(q[0], q[1], x2s, dis, b2.reshape(1, -1))
